# baseline (speedup 1.0000x reference)
import functools

import jax
import jax.numpy as jnp
from jax import lax
from jax.experimental import pallas as pl
from jax.experimental.pallas import tpu as pltpu
from jax.experimental.pallas import tpu_sc as plsc

N = 100000
D = 128
G = 512
NC = 2
NS = 16
NW = NC * NS
SPT = G // NW
C = 448
NEG = -1e30


def _take(v, idx):
    return jnp.take_along_axis(v, idx, axis=0)


def _sc_pool(x_hbm, pb_hbm, bnd_hbm, gw_hbm, out_hbm,
             xv0, xv1, pv0, pv1, bndv, gwv, xsc, sv2, sems):
    wid = lax.axis_index("c") * NS + lax.axis_index("s")
    xbufs, pbufs = [xv0, xv1], [pv0, pv1]

    pltpu.sync_copy(bnd_hbm.at[wid], bndv)
    pltpu.sync_copy(gw_hbm, gwv)
    gws = [gwv[pl.ds(16 * j, 16)] for j in range(8)]

    z16 = jnp.zeros((16,), jnp.float32)
    for sl in range(SPT):
        for j in range(8):
            xsc[sl, pl.ds(16 * j, 16)] = z16
        sv2[sl, :] = z16

    bvec = bndv[pl.ds(0, 16)]
    r0 = bvec[0]
    r1 = bvec[1]
    a0 = (r0 // 8) * 8
    T = (r1 - a0 + C - 1) // C
    Thalf = (T + 1) // 2

    neg_v = jnp.full((16,), NEG, jnp.float32)
    iota = jnp.arange(16, dtype=jnp.int32)
    ones_v = jnp.full((16,), 1.0, jnp.float32)

    def chunk_start(k):
        return jnp.minimum(a0 + k * C, N - C)

    def issue(k, slot):
        s = chunk_start(k)
        pltpu.async_copy(x_hbm.at[pl.ds(s, C), :], xbufs[slot], sems.at[slot, 0])
        pltpu.async_copy(pb_hbm.at[pl.ds(s, C)], pbufs[slot], sems.at[slot, 1])

    def wait(slot):
        pltpu.make_async_copy(x_hbm.at[pl.ds(0, C), :], xbufs[slot],
                              sems.at[slot, 0]).wait()
        pltpu.make_async_copy(pb_hbm.at[pl.ds(0, C)], pbufs[slot],
                              sems.at[slot, 1]).wait()

    def seg_scan_max(x, seg):
        for k in (1, 2, 4, 8):
            src = jnp.maximum(iota - k, 0)
            ok = (iota >= k) & (_take(seg, src) == seg)
            x = jnp.where(ok, jnp.maximum(x, _take(x, src)), x)
        return x

    def seg_fill_back_max(x, seg):
        for k in (1, 2, 4, 8):
            src = jnp.minimum(iota + k, 15)
            ok = (iota + k <= 15) & (_take(seg, src) == seg)
            x = jnp.where(ok, jnp.maximum(x, _take(x, src)), x)
        return x

    def seg_scan_sum(x, seg):
        for k in (1, 2, 4, 8):
            src = jnp.maximum(iota - k, 0)
            ok = (iota >= k) & (_take(seg, src) == seg)
            x = jnp.where(ok, x + _take(x, src), x)
        return x

    def group_body(start_k, lo, slot):
        def body(gidx, carry):
            cur_seg, m_c, s_c, acc = carry
            i0 = gidx * 16
            rbase = start_k + i0

            pk = pbufs[slot][pl.ds(i0, 16)]
            seg_raw = pk >> 1
            mkf = (pk & 1).astype(jnp.float32)

            rvec = rbase + iota
            valid = (rvec >= lo) & (rvec < r1)

            fvi = jnp.min(jnp.where(valid, iota, 16))
            lvi = jnp.max(jnp.where(valid, iota, -1))
            slv_vec = _take(seg_raw, jnp.full((16,), jnp.maximum(lvi, 0),
                                              jnp.int32))
            cur_seg_v = jnp.full((16,), cur_seg, jnp.int32)
            seg_eff = jnp.where(valid, seg_raw,
                                jnp.where(iota < fvi, cur_seg_v, slv_vec))

            prev = jnp.where(iota == 0, cur_seg_v,
                             _take(seg_eff, jnp.maximum(iota - 1, 0)))
            run_start = seg_eff != prev
            keepf = jnp.where(run_start, 0.0, 1.0)

            g_vec = z16
            for l in range(16):
                i = i0 + l
                xr = [xbufs[slot][i, pl.ds(16 * j, 16)] for j in range(8)]
                p0 = ((xr[0] * gws[0] + xr[1] * gws[1])
                      + (xr[2] * gws[2] + xr[3] * gws[3]))
                p1 = ((xr[4] * gws[4] + xr[5] * gws[5])
                      + (xr[6] * gws[6] + xr[7] * gws[7]))
                g = jnp.sum(p0 + p1)
                g_vec = jnp.where(iota == l, g, g_vec)

            mk_on = (mkf > 0.0) & valid
            gm = jnp.where(mk_on, g_vec, neg_v)

            m_f = seg_scan_max(gm, seg_eff)
            in_carry = seg_eff == cur_seg_v
            m_f = jnp.where(in_carry, jnp.maximum(m_f, m_c), m_f)
            m_b = seg_fill_back_max(m_f, seg_eff)

            e_vec = jnp.exp(gm - m_b) * jnp.where(mk_on, ones_v, 0.0)

            mb0 = jnp.full((16,), m_b[0], jnp.float32)
            lane0_carry = seg_eff[0] == cur_seg
            fc = jnp.exp(jnp.where(lane0_carry, m_c - mb0, neg_v))

            s_run = seg_fill_back_max(seg_scan_sum(e_vec, seg_eff), seg_eff)
            s_fin = s_run + jnp.where(in_carry, s_c * fc, 0.0)

            acc = [a * fc for a in acc]
            for l in range(16):
                lane = jnp.full((16,), l, jnp.int32)
                e_l = _take(e_vec, lane)
                k_l = _take(keepf, lane)
                i = i0 + l
                xr = [xbufs[slot][i, pl.ds(16 * j, 16)] for j in range(8)]
                acc = [acc[j] * k_l + e_l * xr[j] for j in range(8)]
                local = jnp.clip(seg_eff[l] - SPT * wid, 0, SPT - 1)
                for j in range(8):
                    xsc[local, pl.ds(16 * j, 16)] = acc[j]
                sv2[local, :] = _take(s_fin, lane)

            return (seg_eff[15], jnp.full((16,), m_b[15], jnp.float32),
                    jnp.full((16,), s_fin[15], jnp.float32), tuple(acc))
        return body

    def process(k, slot, carry):
        cur_seg, m_c, s_c, acc, pe = carry
        start_k = chunk_start(k)
        lo = jnp.maximum(r0, pe)
        inner = lax.fori_loop(0, C // 16, group_body(start_k, lo, slot),
                              (cur_seg, m_c, s_c, acc))
        return inner + (start_k + C,)

    @pl.when(T > 0)
    def _prime():
        issue(0, 0)

    init = (jnp.int32(-1), neg_v, z16, tuple(z16 for _ in range(8)),
            jnp.int32(0))

    def pair_body(kk, carry):
        k0 = 2 * kk
        k1 = 2 * kk + 1

        @pl.when(k1 < T)
        def _():
            issue(k1, 1)
        wait(0)
        carry = process(k0, 0, carry)

        @pl.when(k1 + 1 < T)
        def _():
            issue(k1 + 1, 0)

        def do_k1(c):
            wait(1)
            return process(k1, 1, c)
        carry = lax.cond(k1 < T, do_k1, lambda c: c, carry)
        return carry

    lax.fori_loop(0, Thalf, pair_body, init)

    for sl in range(SPT):
        inv_v = 1.0 / (sv2[sl, pl.ds(0, 16)] + 1e-16)
        for j in range(8):
            xsc[sl, pl.ds(16 * j, 16)] = xsc[sl, pl.ds(16 * j, 16)] * inv_v
    pltpu.sync_copy(xsc, out_hbm.at[pl.ds(wid * SPT, SPT), :])


@functools.partial(
    pl.kernel,
    out_type=jax.ShapeDtypeStruct((G, D), jnp.float32),
    mesh=plsc.VectorSubcoreMesh(core_axis_name="c", subcore_axis_name="s"),
    scratch_types=[
        pltpu.VMEM((C, D), jnp.float32),
        pltpu.VMEM((C, D), jnp.float32),
        pltpu.VMEM((C,), jnp.int32),
        pltpu.VMEM((C,), jnp.int32),
        pltpu.VMEM((16,), jnp.int32),
        pltpu.VMEM((D,), jnp.float32),
        pltpu.VMEM((SPT, D), jnp.float32),
        pltpu.VMEM((SPT, 16), jnp.float32),
        pltpu.SemaphoreType.DMA((2, 2)),
    ],
    compiler_params=pltpu.CompilerParams(needs_layout_passes=False),
)
def _sc_pool_kernel(x_hbm, pb_hbm, bnd_hbm, gw_hbm, out_hbm,
                    xv0, xv1, pv0, pv1, bndv, gwv, xsc, sv2, sems):
    _sc_pool(x_hbm, pb_hbm, bnd_hbm, gw_hbm, out_hbm,
             xv0, xv1, pv0, pv1, bndv, gwv, xsc, sv2, sems)


def _heads_kernel(xs_ref, w1_ref, b1_ref, w2_ref, b2_ref,
                  w3_ref, b3_ref, w4_ref, b4_ref, de_ref, den_ref,
                  out1_ref, out2_ref):
    xs = xs_ref[...]
    h1 = jnp.maximum(jnp.dot(xs, w1_ref[...],
                             preferred_element_type=jnp.float32)
                     + b1_ref[0, :], 0.0)
    v_vec = jnp.dot(h1, w2_ref[...],
                    preferred_element_type=jnp.float32) + b2_ref[0, :]
    h2 = jnp.maximum(jnp.dot(xs, w3_ref[...],
                             preferred_element_type=jnp.float32)
                     + b3_ref[0, :], 0.0)
    v_norm = jnp.dot(h2, w4_ref[...],
                     preferred_element_type=jnp.float32) + b4_ref[0, :]
    de_mean = jnp.mean(de_ref[...], axis=0)
    den_mean = jnp.mean(den_ref[...], axis=0)
    out1_ref[...] = v_vec * de_mean[None, :]
    out2_ref[...] = v_norm * den_mean[None, :]


@jax.jit
def _run(x_clu, pb, bnds, gw,
         W1, b1, W2, b2, W3, b3, W4, b4, de, den):
    x_scene = _sc_pool_kernel(x_clu, pb, bnds, gw)
    return pl.pallas_call(
        _heads_kernel,
        out_shape=[jax.ShapeDtypeStruct((G, 6), jnp.float32),
                   jax.ShapeDtypeStruct((G, 1), jnp.float32)],
    )(x_scene, W1, b1.reshape(1, D), W2, b2.reshape(1, 6),
      W3, b3.reshape(1, D), W4, b4.reshape(1, 1), de, den)


def kernel(x_clu, mask_clu, batch_clu, dist_embedding, dist_embedding_norm,
           gate_W, gate_b, W1, b1, W2, b2, W3, b3, W4, b4):
    batchi = batch_clu.astype(jnp.int32)
    pb = batchi * 2 + mask_clu.astype(jnp.int32)
    qs = jnp.arange(NW + 1, dtype=jnp.int32) * SPT
    b33 = jnp.sum((batchi[:, None] < qs[None, :]).astype(jnp.int32),
                  axis=0).astype(jnp.int32)
    bnds = jnp.zeros((NW, 16), jnp.int32)
    bnds = bnds.at[:, 0].set(b33[:NW]).at[:, 1].set(b33[1:])
    out1, out2 = _run(x_clu, pb, bnds, gate_W[:, 0],
                      W1, b1, W2, b2, W3, b3, W4, b4,
                      dist_embedding, dist_embedding_norm)
    return out1, out2


# speedup vs baseline: 1.0291x; 1.0291x over previous
import functools

import jax
import jax.numpy as jnp
from jax import lax
from jax.experimental import pallas as pl
from jax.experimental.pallas import tpu as pltpu
from jax.experimental.pallas import tpu_sc as plsc

N = 100000
D = 128
G = 512
NC = 2
NS = 16
NW = NC * NS
SPT = G // NW
C = 384
NEG = -1e30


def _take(v, idx):
    return jnp.take_along_axis(v, idx, axis=0)


def _sc_pool(x_hbm, pb_hbm, bnd_hbm, gw_hbm, out_hbm,
             xv0, xv1, pv0, pv1, bndv, gwv, xsc, sv2, sems):
    wid = lax.axis_index("c") * NS + lax.axis_index("s")
    xbufs, pbufs = [xv0, xv1], [pv0, pv1]

    pltpu.sync_copy(bnd_hbm.at[wid], bndv)
    pltpu.sync_copy(gw_hbm, gwv)
    gws = [gwv[pl.ds(16 * j, 16)] for j in range(8)]

    z16 = jnp.zeros((16,), jnp.float32)
    for sl in range(SPT):
        for j in range(8):
            xsc[sl, pl.ds(16 * j, 16)] = z16
        sv2[sl, :] = z16

    bvec = bndv[pl.ds(0, 16)]
    r0 = bvec[0]
    r1 = bvec[1]
    a0 = (r0 // 8) * 8
    T = (r1 - a0 + C - 1) // C
    Thalf = (T + 1) // 2

    neg_v = jnp.full((16,), NEG, jnp.float32)
    iota = jnp.arange(16, dtype=jnp.int32)
    ones_v = jnp.full((16,), 1.0, jnp.float32)

    def chunk_start(k):
        return jnp.minimum(a0 + k * C, N - C)

    def issue(k, slot):
        s = chunk_start(k)
        pltpu.async_copy(x_hbm.at[pl.ds(s, C), :], xbufs[slot], sems.at[slot, 0])
        pltpu.async_copy(pb_hbm.at[pl.ds(s, C)], pbufs[slot], sems.at[slot, 1])

    def wait(slot):
        pltpu.make_async_copy(x_hbm.at[pl.ds(0, C), :], xbufs[slot],
                              sems.at[slot, 0]).wait()
        pltpu.make_async_copy(pb_hbm.at[pl.ds(0, C)], pbufs[slot],
                              sems.at[slot, 1]).wait()

    def seg_scan_max(x, seg):
        for k in (1, 2, 4, 8):
            src = jnp.maximum(iota - k, 0)
            ok = (iota >= k) & (_take(seg, src) == seg)
            x = jnp.where(ok, jnp.maximum(x, _take(x, src)), x)
        return x

    def seg_fill_back_max(x, seg):
        for k in (1, 2, 4, 8):
            src = jnp.minimum(iota + k, 15)
            ok = (iota + k <= 15) & (_take(seg, src) == seg)
            x = jnp.where(ok, jnp.maximum(x, _take(x, src)), x)
        return x

    def seg_scan_sum(x, seg):
        for k in (1, 2, 4, 8):
            src = jnp.maximum(iota - k, 0)
            ok = (iota >= k) & (_take(seg, src) == seg)
            x = jnp.where(ok, x + _take(x, src), x)
        return x

    def group_body(start_k, lo, slot):
        def body(gidx, carry):
            cur_seg, m_c, s_c, acc = carry
            i0 = gidx * 16
            rbase = start_k + i0

            pk = pbufs[slot][pl.ds(i0, 16)]
            seg_raw = pk >> 1
            mkf = (pk & 1).astype(jnp.float32)

            rvec = rbase + iota
            valid = (rvec >= lo) & (rvec < r1)

            fvi = jnp.min(jnp.where(valid, iota, 16))
            lvi = jnp.max(jnp.where(valid, iota, -1))
            slv_vec = _take(seg_raw, jnp.full((16,), jnp.maximum(lvi, 0),
                                              jnp.int32))
            cur_seg_v = jnp.full((16,), cur_seg, jnp.int32)
            seg_eff = jnp.where(valid, seg_raw,
                                jnp.where(iota < fvi, cur_seg_v, slv_vec))

            prev = jnp.where(iota == 0, cur_seg_v,
                             _take(seg_eff, jnp.maximum(iota - 1, 0)))
            run_start = seg_eff != prev
            keepf = jnp.where(run_start, 0.0, 1.0)

            g_vec = z16
            for l in range(16):
                i = i0 + l
                xr = [xbufs[slot][i, pl.ds(16 * j, 16)] for j in range(8)]
                p0 = ((xr[0] * gws[0] + xr[1] * gws[1])
                      + (xr[2] * gws[2] + xr[3] * gws[3]))
                p1 = ((xr[4] * gws[4] + xr[5] * gws[5])
                      + (xr[6] * gws[6] + xr[7] * gws[7]))
                g = jnp.sum(p0 + p1)
                g_vec = jnp.where(iota == l, g, g_vec)

            mk_on = (mkf > 0.0) & valid
            gm = jnp.where(mk_on, g_vec, neg_v)

            m_f = seg_scan_max(gm, seg_eff)
            in_carry = seg_eff == cur_seg_v
            m_f = jnp.where(in_carry, jnp.maximum(m_f, m_c), m_f)
            m_b = seg_fill_back_max(m_f, seg_eff)

            e_vec = jnp.exp(gm - m_b) * jnp.where(mk_on, ones_v, 0.0)

            mb0 = jnp.full((16,), m_b[0], jnp.float32)
            lane0_carry = seg_eff[0] == cur_seg
            fc = jnp.exp(jnp.where(lane0_carry, m_c - mb0, neg_v))

            s_run = seg_fill_back_max(seg_scan_sum(e_vec, seg_eff), seg_eff)
            s_fin = s_run + jnp.where(in_carry, s_c * fc, 0.0)

            acc = [a * fc for a in acc]
            for l in range(16):
                lane = jnp.full((16,), l, jnp.int32)
                e_l = _take(e_vec, lane)
                k_l = _take(keepf, lane)
                i = i0 + l
                xr = [xbufs[slot][i, pl.ds(16 * j, 16)] for j in range(8)]
                acc = [acc[j] * k_l + e_l * xr[j] for j in range(8)]
                local = jnp.clip(seg_eff[l] - SPT * wid, 0, SPT - 1)
                for j in range(8):
                    xsc[local, pl.ds(16 * j, 16)] = acc[j]
                sv2[local, :] = _take(s_fin, lane)

            return (seg_eff[15], jnp.full((16,), m_b[15], jnp.float32),
                    jnp.full((16,), s_fin[15], jnp.float32), tuple(acc))
        return body

    def process(k, slot, carry):
        cur_seg, m_c, s_c, acc, pe = carry
        start_k = chunk_start(k)
        lo = jnp.maximum(r0, pe)
        inner = lax.fori_loop(0, C // 16, group_body(start_k, lo, slot),
                              (cur_seg, m_c, s_c, acc))
        return inner + (start_k + C,)

    @pl.when(T > 0)
    def _prime():
        issue(0, 0)

    init = (jnp.int32(-1), neg_v, z16, tuple(z16 for _ in range(8)),
            jnp.int32(0))

    def pair_body(kk, carry):
        k0 = 2 * kk
        k1 = 2 * kk + 1

        @pl.when(k1 < T)
        def _():
            issue(k1, 1)
        wait(0)
        carry = process(k0, 0, carry)

        @pl.when(k1 + 1 < T)
        def _():
            issue(k1 + 1, 0)

        def do_k1(c):
            wait(1)
            return process(k1, 1, c)
        carry = lax.cond(k1 < T, do_k1, lambda c: c, carry)
        return carry

    lax.fori_loop(0, Thalf, pair_body, init)

    for sl in range(SPT):
        inv_v = 1.0 / (sv2[sl, pl.ds(0, 16)] + 1e-16)
        for j in range(8):
            xsc[sl, pl.ds(16 * j, 16)] = xsc[sl, pl.ds(16 * j, 16)] * inv_v
    pltpu.sync_copy(xsc, out_hbm.at[pl.ds(wid * SPT, SPT), :])


@functools.partial(
    pl.kernel,
    out_type=jax.ShapeDtypeStruct((G, D), jnp.float32),
    mesh=plsc.VectorSubcoreMesh(core_axis_name="c", subcore_axis_name="s"),
    scratch_types=[
        pltpu.VMEM((C, D), jnp.float32),
        pltpu.VMEM((C, D), jnp.float32),
        pltpu.VMEM((C,), jnp.int32),
        pltpu.VMEM((C,), jnp.int32),
        pltpu.VMEM((16,), jnp.int32),
        pltpu.VMEM((D,), jnp.float32),
        pltpu.VMEM((SPT, D), jnp.float32),
        pltpu.VMEM((SPT, 16), jnp.float32),
        pltpu.SemaphoreType.DMA((2, 2)),
    ],
    compiler_params=pltpu.CompilerParams(needs_layout_passes=False),
)
def _sc_pool_kernel(x_hbm, pb_hbm, bnd_hbm, gw_hbm, out_hbm,
                    xv0, xv1, pv0, pv1, bndv, gwv, xsc, sv2, sems):
    _sc_pool(x_hbm, pb_hbm, bnd_hbm, gw_hbm, out_hbm,
             xv0, xv1, pv0, pv1, bndv, gwv, xsc, sv2, sems)


def _heads_kernel(xs_ref, w1_ref, b1_ref, w2_ref, b2_ref,
                  w3_ref, b3_ref, w4_ref, b4_ref, de_ref, den_ref,
                  out1_ref, out2_ref):
    xs = xs_ref[...]
    h1 = jnp.maximum(jnp.dot(xs, w1_ref[...],
                             preferred_element_type=jnp.float32)
                     + b1_ref[0, :], 0.0)
    v_vec = jnp.dot(h1, w2_ref[...],
                    preferred_element_type=jnp.float32) + b2_ref[0, :]
    h2 = jnp.maximum(jnp.dot(xs, w3_ref[...],
                             preferred_element_type=jnp.float32)
                     + b3_ref[0, :], 0.0)
    v_norm = jnp.dot(h2, w4_ref[...],
                     preferred_element_type=jnp.float32) + b4_ref[0, :]
    de_mean = jnp.mean(de_ref[...], axis=0)
    den_mean = jnp.mean(den_ref[...], axis=0)
    out1_ref[...] = v_vec * de_mean[None, :]
    out2_ref[...] = v_norm * den_mean[None, :]


@jax.jit
def _run(x_clu, pb, bnds, gw,
         W1, b1, W2, b2, W3, b3, W4, b4, de, den):
    x_scene = _sc_pool_kernel(x_clu, pb, bnds, gw)
    return pl.pallas_call(
        _heads_kernel,
        out_shape=[jax.ShapeDtypeStruct((G, 6), jnp.float32),
                   jax.ShapeDtypeStruct((G, 1), jnp.float32)],
    )(x_scene, W1, b1.reshape(1, D), W2, b2.reshape(1, 6),
      W3, b3.reshape(1, D), W4, b4.reshape(1, 1), de, den)


def kernel(x_clu, mask_clu, batch_clu, dist_embedding, dist_embedding_norm,
           gate_W, gate_b, W1, b1, W2, b2, W3, b3, W4, b4):
    batchi = batch_clu.astype(jnp.int32)
    pb = batchi * 2 + mask_clu.astype(jnp.int32)
    qs = jnp.arange(NW + 1, dtype=jnp.int32) * SPT
    b33 = jnp.sum((batchi[:, None] < qs[None, :]).astype(jnp.int32),
                  axis=0).astype(jnp.int32)
    bnds = jnp.zeros((NW, 16), jnp.int32)
    bnds = bnds.at[:, 0].set(b33[:NW]).at[:, 1].set(b33[1:])
    out1, out2 = _run(x_clu, pb, bnds, gate_W[:, 0],
                      W1, b1, W2, b2, W3, b3, W4, b4,
                      dist_embedding, dist_embedding_norm)
    return out1, out2


# speedup vs baseline: 1.0337x; 1.0045x over previous
import functools

import jax
import jax.numpy as jnp
from jax import lax
from jax.experimental import pallas as pl
from jax.experimental.pallas import tpu as pltpu
from jax.experimental.pallas import tpu_sc as plsc

N = 100000
D = 128
G = 512
NC = 2
NS = 16
NW = NC * NS
SPT = G // NW
C = 320
NEG = -1e30


def _take(v, idx):
    return jnp.take_along_axis(v, idx, axis=0)


def _sc_pool(x_hbm, pb_hbm, bnd_hbm, gw_hbm, out_hbm,
             xv0, xv1, pv0, pv1, bndv, gwv, xsc, sv2, sems):
    wid = lax.axis_index("c") * NS + lax.axis_index("s")
    xbufs, pbufs = [xv0, xv1], [pv0, pv1]

    pltpu.sync_copy(bnd_hbm.at[wid], bndv)
    pltpu.sync_copy(gw_hbm, gwv)
    gws = [gwv[pl.ds(16 * j, 16)] for j in range(8)]

    z16 = jnp.zeros((16,), jnp.float32)
    for sl in range(SPT):
        for j in range(8):
            xsc[sl, pl.ds(16 * j, 16)] = z16
        sv2[sl, :] = z16

    bvec = bndv[pl.ds(0, 16)]
    r0 = bvec[0]
    r1 = bvec[1]
    a0 = (r0 // 8) * 8
    T = (r1 - a0 + C - 1) // C
    Thalf = (T + 1) // 2

    neg_v = jnp.full((16,), NEG, jnp.float32)
    iota = jnp.arange(16, dtype=jnp.int32)
    ones_v = jnp.full((16,), 1.0, jnp.float32)

    def chunk_start(k):
        return jnp.minimum(a0 + k * C, N - C)

    def issue(k, slot):
        s = chunk_start(k)
        pltpu.async_copy(x_hbm.at[pl.ds(s, C), :], xbufs[slot], sems.at[slot, 0])
        pltpu.async_copy(pb_hbm.at[pl.ds(s, C)], pbufs[slot], sems.at[slot, 1])

    def wait(slot):
        pltpu.make_async_copy(x_hbm.at[pl.ds(0, C), :], xbufs[slot],
                              sems.at[slot, 0]).wait()
        pltpu.make_async_copy(pb_hbm.at[pl.ds(0, C)], pbufs[slot],
                              sems.at[slot, 1]).wait()

    def seg_scan_max(x, seg):
        for k in (1, 2, 4, 8):
            src = jnp.maximum(iota - k, 0)
            ok = (iota >= k) & (_take(seg, src) == seg)
            x = jnp.where(ok, jnp.maximum(x, _take(x, src)), x)
        return x

    def seg_fill_back_max(x, seg):
        for k in (1, 2, 4, 8):
            src = jnp.minimum(iota + k, 15)
            ok = (iota + k <= 15) & (_take(seg, src) == seg)
            x = jnp.where(ok, jnp.maximum(x, _take(x, src)), x)
        return x

    def seg_scan_sum(x, seg):
        for k in (1, 2, 4, 8):
            src = jnp.maximum(iota - k, 0)
            ok = (iota >= k) & (_take(seg, src) == seg)
            x = jnp.where(ok, x + _take(x, src), x)
        return x

    def group_body(start_k, lo, slot):
        def body(gidx, carry):
            cur_seg, m_c, s_c, acc = carry
            i0 = gidx * 16
            rbase = start_k + i0

            pk = pbufs[slot][pl.ds(i0, 16)]
            seg_raw = pk >> 1
            mkf = (pk & 1).astype(jnp.float32)

            rvec = rbase + iota
            valid = (rvec >= lo) & (rvec < r1)

            fvi = jnp.min(jnp.where(valid, iota, 16))
            lvi = jnp.max(jnp.where(valid, iota, -1))
            slv_vec = _take(seg_raw, jnp.full((16,), jnp.maximum(lvi, 0),
                                              jnp.int32))
            cur_seg_v = jnp.full((16,), cur_seg, jnp.int32)
            seg_eff = jnp.where(valid, seg_raw,
                                jnp.where(iota < fvi, cur_seg_v, slv_vec))

            prev = jnp.where(iota == 0, cur_seg_v,
                             _take(seg_eff, jnp.maximum(iota - 1, 0)))
            run_start = seg_eff != prev
            keepf = jnp.where(run_start, 0.0, 1.0)

            g_vec = z16
            for l in range(16):
                i = i0 + l
                xr = [xbufs[slot][i, pl.ds(16 * j, 16)] for j in range(8)]
                p0 = ((xr[0] * gws[0] + xr[1] * gws[1])
                      + (xr[2] * gws[2] + xr[3] * gws[3]))
                p1 = ((xr[4] * gws[4] + xr[5] * gws[5])
                      + (xr[6] * gws[6] + xr[7] * gws[7]))
                g = jnp.sum(p0 + p1)
                g_vec = jnp.where(iota == l, g, g_vec)

            mk_on = (mkf > 0.0) & valid
            gm = jnp.where(mk_on, g_vec, neg_v)

            m_f = seg_scan_max(gm, seg_eff)
            in_carry = seg_eff == cur_seg_v
            m_f = jnp.where(in_carry, jnp.maximum(m_f, m_c), m_f)
            m_b = seg_fill_back_max(m_f, seg_eff)

            e_vec = jnp.exp(gm - m_b) * jnp.where(mk_on, ones_v, 0.0)

            mb0 = jnp.full((16,), m_b[0], jnp.float32)
            lane0_carry = seg_eff[0] == cur_seg
            fc = jnp.exp(jnp.where(lane0_carry, m_c - mb0, neg_v))

            s_run = seg_fill_back_max(seg_scan_sum(e_vec, seg_eff), seg_eff)
            s_fin = s_run + jnp.where(in_carry, s_c * fc, 0.0)

            acc = [a * fc for a in acc]
            for l in range(16):
                lane = jnp.full((16,), l, jnp.int32)
                e_l = _take(e_vec, lane)
                k_l = _take(keepf, lane)
                i = i0 + l
                xr = [xbufs[slot][i, pl.ds(16 * j, 16)] for j in range(8)]
                acc = [acc[j] * k_l + e_l * xr[j] for j in range(8)]
                local = jnp.clip(seg_eff[l] - SPT * wid, 0, SPT - 1)
                for j in range(8):
                    xsc[local, pl.ds(16 * j, 16)] = acc[j]
                sv2[local, :] = _take(s_fin, lane)

            return (seg_eff[15], jnp.full((16,), m_b[15], jnp.float32),
                    jnp.full((16,), s_fin[15], jnp.float32), tuple(acc))
        return body

    def process(k, slot, carry):
        cur_seg, m_c, s_c, acc, pe = carry
        start_k = chunk_start(k)
        lo = jnp.maximum(r0, pe)
        inner = lax.fori_loop(0, C // 16, group_body(start_k, lo, slot),
                              (cur_seg, m_c, s_c, acc))
        return inner + (start_k + C,)

    @pl.when(T > 0)
    def _prime():
        issue(0, 0)

    init = (jnp.int32(-1), neg_v, z16, tuple(z16 for _ in range(8)),
            jnp.int32(0))

    def pair_body(kk, carry):
        k0 = 2 * kk
        k1 = 2 * kk + 1

        @pl.when(k1 < T)
        def _():
            issue(k1, 1)
        wait(0)
        carry = process(k0, 0, carry)

        @pl.when(k1 + 1 < T)
        def _():
            issue(k1 + 1, 0)

        def do_k1(c):
            wait(1)
            return process(k1, 1, c)
        carry = lax.cond(k1 < T, do_k1, lambda c: c, carry)
        return carry

    lax.fori_loop(0, Thalf, pair_body, init)

    for sl in range(SPT):
        inv_v = 1.0 / (sv2[sl, pl.ds(0, 16)] + 1e-16)
        for j in range(8):
            xsc[sl, pl.ds(16 * j, 16)] = xsc[sl, pl.ds(16 * j, 16)] * inv_v
    pltpu.sync_copy(xsc, out_hbm.at[pl.ds(wid * SPT, SPT), :])


@functools.partial(
    pl.kernel,
    out_type=jax.ShapeDtypeStruct((G, D), jnp.float32),
    mesh=plsc.VectorSubcoreMesh(core_axis_name="c", subcore_axis_name="s"),
    scratch_types=[
        pltpu.VMEM((C, D), jnp.float32),
        pltpu.VMEM((C, D), jnp.float32),
        pltpu.VMEM((C,), jnp.int32),
        pltpu.VMEM((C,), jnp.int32),
        pltpu.VMEM((16,), jnp.int32),
        pltpu.VMEM((D,), jnp.float32),
        pltpu.VMEM((SPT, D), jnp.float32),
        pltpu.VMEM((SPT, 16), jnp.float32),
        pltpu.SemaphoreType.DMA((2, 2)),
    ],
    compiler_params=pltpu.CompilerParams(needs_layout_passes=False),
)
def _sc_pool_kernel(x_hbm, pb_hbm, bnd_hbm, gw_hbm, out_hbm,
                    xv0, xv1, pv0, pv1, bndv, gwv, xsc, sv2, sems):
    _sc_pool(x_hbm, pb_hbm, bnd_hbm, gw_hbm, out_hbm,
             xv0, xv1, pv0, pv1, bndv, gwv, xsc, sv2, sems)


def _heads_kernel(xs_ref, w1_ref, b1_ref, w2_ref, b2_ref,
                  w3_ref, b3_ref, w4_ref, b4_ref, de_ref, den_ref,
                  out1_ref, out2_ref):
    xs = xs_ref[...]
    h1 = jnp.maximum(jnp.dot(xs, w1_ref[...],
                             preferred_element_type=jnp.float32)
                     + b1_ref[0, :], 0.0)
    v_vec = jnp.dot(h1, w2_ref[...],
                    preferred_element_type=jnp.float32) + b2_ref[0, :]
    h2 = jnp.maximum(jnp.dot(xs, w3_ref[...],
                             preferred_element_type=jnp.float32)
                     + b3_ref[0, :], 0.0)
    v_norm = jnp.dot(h2, w4_ref[...],
                     preferred_element_type=jnp.float32) + b4_ref[0, :]
    de_mean = jnp.mean(de_ref[...], axis=0)
    den_mean = jnp.mean(den_ref[...], axis=0)
    out1_ref[...] = v_vec * de_mean[None, :]
    out2_ref[...] = v_norm * den_mean[None, :]


@jax.jit
def _run(x_clu, pb, bnds, gw,
         W1, b1, W2, b2, W3, b3, W4, b4, de, den):
    x_scene = _sc_pool_kernel(x_clu, pb, bnds, gw)
    return pl.pallas_call(
        _heads_kernel,
        out_shape=[jax.ShapeDtypeStruct((G, 6), jnp.float32),
                   jax.ShapeDtypeStruct((G, 1), jnp.float32)],
    )(x_scene, W1, b1.reshape(1, D), W2, b2.reshape(1, 6),
      W3, b3.reshape(1, D), W4, b4.reshape(1, 1), de, den)


def kernel(x_clu, mask_clu, batch_clu, dist_embedding, dist_embedding_norm,
           gate_W, gate_b, W1, b1, W2, b2, W3, b3, W4, b4):
    batchi = batch_clu.astype(jnp.int32)
    pb = batchi * 2 + mask_clu.astype(jnp.int32)
    qs = jnp.arange(NW + 1, dtype=jnp.int32) * SPT
    b33 = jnp.sum((batchi[:, None] < qs[None, :]).astype(jnp.int32),
                  axis=0).astype(jnp.int32)
    bnds = jnp.zeros((NW, 16), jnp.int32)
    bnds = bnds.at[:, 0].set(b33[:NW]).at[:, 1].set(b33[1:])
    out1, out2 = _run(x_clu, pb, bnds, gate_W[:, 0],
                      W1, b1, W2, b2, W3, b3, W4, b4,
                      dist_embedding, dist_embedding_norm)
    return out1, out2


# speedup vs baseline: 1.0688x; 1.0340x over previous
import functools

import jax
import jax.numpy as jnp
from jax import lax
from jax.experimental import pallas as pl
from jax.experimental.pallas import tpu as pltpu
from jax.experimental.pallas import tpu_sc as plsc

N = 100000
D = 128
G = 512
NC = 2
NS = 16
NW = NC * NS
SPT = G // NW
C = 256
NEG = -1e30


def _take(v, idx):
    return jnp.take_along_axis(v, idx, axis=0)


def _sc_pool(x_hbm, pb_hbm, bnd_hbm, gw_hbm, out_hbm,
             xv0, xv1, pv0, pv1, bndv, gwv, xsc, sv2, sems):
    wid = lax.axis_index("c") * NS + lax.axis_index("s")
    xbufs, pbufs = [xv0, xv1], [pv0, pv1]

    pltpu.sync_copy(bnd_hbm.at[wid], bndv)
    pltpu.sync_copy(gw_hbm, gwv)
    gws = [gwv[pl.ds(16 * j, 16)] for j in range(8)]

    z16 = jnp.zeros((16,), jnp.float32)
    for sl in range(SPT):
        for j in range(8):
            xsc[sl, pl.ds(16 * j, 16)] = z16
        sv2[sl, :] = z16

    bvec = bndv[pl.ds(0, 16)]
    r0 = bvec[0]
    r1 = bvec[1]
    a0 = (r0 // 8) * 8
    T = (r1 - a0 + C - 1) // C
    Thalf = (T + 1) // 2

    neg_v = jnp.full((16,), NEG, jnp.float32)
    iota = jnp.arange(16, dtype=jnp.int32)
    ones_v = jnp.full((16,), 1.0, jnp.float32)

    def chunk_start(k):
        return jnp.minimum(a0 + k * C, N - C)

    def issue(k, slot):
        s = chunk_start(k)
        pltpu.async_copy(x_hbm.at[pl.ds(s, C), :], xbufs[slot], sems.at[slot, 0])
        pltpu.async_copy(pb_hbm.at[pl.ds(s, C)], pbufs[slot], sems.at[slot, 1])

    def wait(slot):
        pltpu.make_async_copy(x_hbm.at[pl.ds(0, C), :], xbufs[slot],
                              sems.at[slot, 0]).wait()
        pltpu.make_async_copy(pb_hbm.at[pl.ds(0, C)], pbufs[slot],
                              sems.at[slot, 1]).wait()

    def seg_scan_max(x, seg):
        for k in (1, 2, 4, 8):
            src = jnp.maximum(iota - k, 0)
            ok = (iota >= k) & (_take(seg, src) == seg)
            x = jnp.where(ok, jnp.maximum(x, _take(x, src)), x)
        return x

    def seg_fill_back_max(x, seg):
        for k in (1, 2, 4, 8):
            src = jnp.minimum(iota + k, 15)
            ok = (iota + k <= 15) & (_take(seg, src) == seg)
            x = jnp.where(ok, jnp.maximum(x, _take(x, src)), x)
        return x

    def seg_scan_sum(x, seg):
        for k in (1, 2, 4, 8):
            src = jnp.maximum(iota - k, 0)
            ok = (iota >= k) & (_take(seg, src) == seg)
            x = jnp.where(ok, x + _take(x, src), x)
        return x

    def group_body(start_k, lo, slot):
        def body(gidx, carry):
            cur_seg, m_c, s_c, acc = carry
            i0 = gidx * 16
            rbase = start_k + i0

            pk = pbufs[slot][pl.ds(i0, 16)]
            seg_raw = pk >> 1
            mkf = (pk & 1).astype(jnp.float32)

            rvec = rbase + iota
            valid = (rvec >= lo) & (rvec < r1)

            fvi = jnp.min(jnp.where(valid, iota, 16))
            lvi = jnp.max(jnp.where(valid, iota, -1))
            slv_vec = _take(seg_raw, jnp.full((16,), jnp.maximum(lvi, 0),
                                              jnp.int32))
            cur_seg_v = jnp.full((16,), cur_seg, jnp.int32)
            seg_eff = jnp.where(valid, seg_raw,
                                jnp.where(iota < fvi, cur_seg_v, slv_vec))

            prev = jnp.where(iota == 0, cur_seg_v,
                             _take(seg_eff, jnp.maximum(iota - 1, 0)))
            run_start = seg_eff != prev
            keepf = jnp.where(run_start, 0.0, 1.0)

            g_vec = z16
            for l in range(16):
                i = i0 + l
                xr = [xbufs[slot][i, pl.ds(16 * j, 16)] for j in range(8)]
                p0 = ((xr[0] * gws[0] + xr[1] * gws[1])
                      + (xr[2] * gws[2] + xr[3] * gws[3]))
                p1 = ((xr[4] * gws[4] + xr[5] * gws[5])
                      + (xr[6] * gws[6] + xr[7] * gws[7]))
                g = jnp.sum(p0 + p1)
                g_vec = jnp.where(iota == l, g, g_vec)

            mk_on = (mkf > 0.0) & valid
            gm = jnp.where(mk_on, g_vec, neg_v)

            m_f = seg_scan_max(gm, seg_eff)
            in_carry = seg_eff == cur_seg_v
            m_f = jnp.where(in_carry, jnp.maximum(m_f, m_c), m_f)
            m_b = seg_fill_back_max(m_f, seg_eff)

            e_vec = jnp.exp(gm - m_b) * jnp.where(mk_on, ones_v, 0.0)

            mb0 = jnp.full((16,), m_b[0], jnp.float32)
            lane0_carry = seg_eff[0] == cur_seg
            fc = jnp.exp(jnp.where(lane0_carry, m_c - mb0, neg_v))

            s_run = seg_fill_back_max(seg_scan_sum(e_vec, seg_eff), seg_eff)
            s_fin = s_run + jnp.where(in_carry, s_c * fc, 0.0)

            acc = [a * fc for a in acc]
            for l in range(16):
                lane = jnp.full((16,), l, jnp.int32)
                e_l = _take(e_vec, lane)
                k_l = _take(keepf, lane)
                i = i0 + l
                xr = [xbufs[slot][i, pl.ds(16 * j, 16)] for j in range(8)]
                acc = [acc[j] * k_l + e_l * xr[j] for j in range(8)]
                local = jnp.clip(seg_eff[l] - SPT * wid, 0, SPT - 1)
                for j in range(8):
                    xsc[local, pl.ds(16 * j, 16)] = acc[j]
                sv2[local, :] = _take(s_fin, lane)

            return (seg_eff[15], jnp.full((16,), m_b[15], jnp.float32),
                    jnp.full((16,), s_fin[15], jnp.float32), tuple(acc))
        return body

    def process(k, slot, carry):
        cur_seg, m_c, s_c, acc, pe = carry
        start_k = chunk_start(k)
        lo = jnp.maximum(r0, pe)
        inner = lax.fori_loop(0, C // 16, group_body(start_k, lo, slot),
                              (cur_seg, m_c, s_c, acc))
        return inner + (start_k + C,)

    @pl.when(T > 0)
    def _prime():
        issue(0, 0)

    init = (jnp.int32(-1), neg_v, z16, tuple(z16 for _ in range(8)),
            jnp.int32(0))

    def pair_body(kk, carry):
        k0 = 2 * kk
        k1 = 2 * kk + 1

        @pl.when(k1 < T)
        def _():
            issue(k1, 1)
        wait(0)
        carry = process(k0, 0, carry)

        @pl.when(k1 + 1 < T)
        def _():
            issue(k1 + 1, 0)

        def do_k1(c):
            wait(1)
            return process(k1, 1, c)
        carry = lax.cond(k1 < T, do_k1, lambda c: c, carry)
        return carry

    lax.fori_loop(0, Thalf, pair_body, init)

    for sl in range(SPT):
        inv_v = 1.0 / (sv2[sl, pl.ds(0, 16)] + 1e-16)
        for j in range(8):
            xsc[sl, pl.ds(16 * j, 16)] = xsc[sl, pl.ds(16 * j, 16)] * inv_v
    pltpu.sync_copy(xsc, out_hbm.at[pl.ds(wid * SPT, SPT), :])


@functools.partial(
    pl.kernel,
    out_type=jax.ShapeDtypeStruct((G, D), jnp.float32),
    mesh=plsc.VectorSubcoreMesh(core_axis_name="c", subcore_axis_name="s"),
    scratch_types=[
        pltpu.VMEM((C, D), jnp.float32),
        pltpu.VMEM((C, D), jnp.float32),
        pltpu.VMEM((C,), jnp.int32),
        pltpu.VMEM((C,), jnp.int32),
        pltpu.VMEM((16,), jnp.int32),
        pltpu.VMEM((D,), jnp.float32),
        pltpu.VMEM((SPT, D), jnp.float32),
        pltpu.VMEM((SPT, 16), jnp.float32),
        pltpu.SemaphoreType.DMA((2, 2)),
    ],
    compiler_params=pltpu.CompilerParams(needs_layout_passes=False),
)
def _sc_pool_kernel(x_hbm, pb_hbm, bnd_hbm, gw_hbm, out_hbm,
                    xv0, xv1, pv0, pv1, bndv, gwv, xsc, sv2, sems):
    _sc_pool(x_hbm, pb_hbm, bnd_hbm, gw_hbm, out_hbm,
             xv0, xv1, pv0, pv1, bndv, gwv, xsc, sv2, sems)


def _heads_kernel(xs_ref, w1_ref, b1_ref, w2_ref, b2_ref,
                  w3_ref, b3_ref, w4_ref, b4_ref, de_ref, den_ref,
                  out1_ref, out2_ref):
    xs = xs_ref[...]
    h1 = jnp.maximum(jnp.dot(xs, w1_ref[...],
                             preferred_element_type=jnp.float32)
                     + b1_ref[0, :], 0.0)
    v_vec = jnp.dot(h1, w2_ref[...],
                    preferred_element_type=jnp.float32) + b2_ref[0, :]
    h2 = jnp.maximum(jnp.dot(xs, w3_ref[...],
                             preferred_element_type=jnp.float32)
                     + b3_ref[0, :], 0.0)
    v_norm = jnp.dot(h2, w4_ref[...],
                     preferred_element_type=jnp.float32) + b4_ref[0, :]
    de_mean = jnp.mean(de_ref[...], axis=0)
    den_mean = jnp.mean(den_ref[...], axis=0)
    out1_ref[...] = v_vec * de_mean[None, :]
    out2_ref[...] = v_norm * den_mean[None, :]


@jax.jit
def _run(x_clu, pb, bnds, gw,
         W1, b1, W2, b2, W3, b3, W4, b4, de, den):
    x_scene = _sc_pool_kernel(x_clu, pb, bnds, gw)
    return pl.pallas_call(
        _heads_kernel,
        out_shape=[jax.ShapeDtypeStruct((G, 6), jnp.float32),
                   jax.ShapeDtypeStruct((G, 1), jnp.float32)],
    )(x_scene, W1, b1.reshape(1, D), W2, b2.reshape(1, 6),
      W3, b3.reshape(1, D), W4, b4.reshape(1, 1), de, den)


def kernel(x_clu, mask_clu, batch_clu, dist_embedding, dist_embedding_norm,
           gate_W, gate_b, W1, b1, W2, b2, W3, b3, W4, b4):
    batchi = batch_clu.astype(jnp.int32)
    pb = batchi * 2 + mask_clu.astype(jnp.int32)
    qs = jnp.arange(NW + 1, dtype=jnp.int32) * SPT
    b33 = jnp.sum((batchi[:, None] < qs[None, :]).astype(jnp.int32),
                  axis=0).astype(jnp.int32)
    bnds = jnp.zeros((NW, 16), jnp.int32)
    bnds = bnds.at[:, 0].set(b33[:NW]).at[:, 1].set(b33[1:])
    out1, out2 = _run(x_clu, pb, bnds, gate_W[:, 0],
                      W1, b1, W2, b2, W3, b3, W4, b4,
                      dist_embedding, dist_embedding_norm)
    return out1, out2


# speedup vs baseline: 1.0812x; 1.0115x over previous
import functools

import jax
import jax.numpy as jnp
from jax import lax
from jax.experimental import pallas as pl
from jax.experimental.pallas import tpu as pltpu
from jax.experimental.pallas import tpu_sc as plsc

N = 100000
D = 128
G = 512
NC = 2
NS = 16
NW = NC * NS
SPT = G // NW
C = 160
NEG = -1e30


def _take(v, idx):
    return jnp.take_along_axis(v, idx, axis=0)


def _sc_pool(x_hbm, pb_hbm, bnd_hbm, gw_hbm, out_hbm,
             xv0, xv1, pv0, pv1, bndv, gwv, xsc, sv2, sems):
    wid = lax.axis_index("c") * NS + lax.axis_index("s")
    xbufs, pbufs = [xv0, xv1], [pv0, pv1]

    pltpu.sync_copy(bnd_hbm.at[wid], bndv)
    pltpu.sync_copy(gw_hbm, gwv)
    gws = [gwv[pl.ds(16 * j, 16)] for j in range(8)]

    z16 = jnp.zeros((16,), jnp.float32)
    for sl in range(SPT):
        for j in range(8):
            xsc[sl, pl.ds(16 * j, 16)] = z16
        sv2[sl, :] = z16

    bvec = bndv[pl.ds(0, 16)]
    r0 = bvec[0]
    r1 = bvec[1]
    a0 = (r0 // 8) * 8
    T = (r1 - a0 + C - 1) // C
    Thalf = (T + 1) // 2

    neg_v = jnp.full((16,), NEG, jnp.float32)
    iota = jnp.arange(16, dtype=jnp.int32)
    ones_v = jnp.full((16,), 1.0, jnp.float32)

    def chunk_start(k):
        return jnp.minimum(a0 + k * C, N - C)

    def issue(k, slot):
        s = chunk_start(k)
        pltpu.async_copy(x_hbm.at[pl.ds(s, C), :], xbufs[slot], sems.at[slot, 0])
        pltpu.async_copy(pb_hbm.at[pl.ds(s, C)], pbufs[slot], sems.at[slot, 1])

    def wait(slot):
        pltpu.make_async_copy(x_hbm.at[pl.ds(0, C), :], xbufs[slot],
                              sems.at[slot, 0]).wait()
        pltpu.make_async_copy(pb_hbm.at[pl.ds(0, C)], pbufs[slot],
                              sems.at[slot, 1]).wait()

    def seg_scan_max(x, seg):
        for k in (1, 2, 4, 8):
            src = jnp.maximum(iota - k, 0)
            ok = (iota >= k) & (_take(seg, src) == seg)
            x = jnp.where(ok, jnp.maximum(x, _take(x, src)), x)
        return x

    def seg_fill_back_max(x, seg):
        for k in (1, 2, 4, 8):
            src = jnp.minimum(iota + k, 15)
            ok = (iota + k <= 15) & (_take(seg, src) == seg)
            x = jnp.where(ok, jnp.maximum(x, _take(x, src)), x)
        return x

    def seg_scan_sum(x, seg):
        for k in (1, 2, 4, 8):
            src = jnp.maximum(iota - k, 0)
            ok = (iota >= k) & (_take(seg, src) == seg)
            x = jnp.where(ok, x + _take(x, src), x)
        return x

    def group_body(start_k, lo, slot):
        def body(gidx, carry):
            cur_seg, m_c, s_c, acc = carry
            i0 = gidx * 16
            rbase = start_k + i0

            pk = pbufs[slot][pl.ds(i0, 16)]
            seg_raw = pk >> 1
            mkf = (pk & 1).astype(jnp.float32)

            rvec = rbase + iota
            valid = (rvec >= lo) & (rvec < r1)

            fvi = jnp.min(jnp.where(valid, iota, 16))
            lvi = jnp.max(jnp.where(valid, iota, -1))
            slv_vec = _take(seg_raw, jnp.full((16,), jnp.maximum(lvi, 0),
                                              jnp.int32))
            cur_seg_v = jnp.full((16,), cur_seg, jnp.int32)
            seg_eff = jnp.where(valid, seg_raw,
                                jnp.where(iota < fvi, cur_seg_v, slv_vec))

            prev = jnp.where(iota == 0, cur_seg_v,
                             _take(seg_eff, jnp.maximum(iota - 1, 0)))
            run_start = seg_eff != prev
            keepf = jnp.where(run_start, 0.0, 1.0)

            g_vec = z16
            for l in range(16):
                i = i0 + l
                xr = [xbufs[slot][i, pl.ds(16 * j, 16)] for j in range(8)]
                p0 = ((xr[0] * gws[0] + xr[1] * gws[1])
                      + (xr[2] * gws[2] + xr[3] * gws[3]))
                p1 = ((xr[4] * gws[4] + xr[5] * gws[5])
                      + (xr[6] * gws[6] + xr[7] * gws[7]))
                g = jnp.sum(p0 + p1)
                g_vec = jnp.where(iota == l, g, g_vec)

            mk_on = (mkf > 0.0) & valid
            gm = jnp.where(mk_on, g_vec, neg_v)

            m_f = seg_scan_max(gm, seg_eff)
            in_carry = seg_eff == cur_seg_v
            m_f = jnp.where(in_carry, jnp.maximum(m_f, m_c), m_f)
            m_b = seg_fill_back_max(m_f, seg_eff)

            e_vec = jnp.exp(gm - m_b) * jnp.where(mk_on, ones_v, 0.0)

            mb0 = jnp.full((16,), m_b[0], jnp.float32)
            lane0_carry = seg_eff[0] == cur_seg
            fc = jnp.exp(jnp.where(lane0_carry, m_c - mb0, neg_v))

            s_run = seg_fill_back_max(seg_scan_sum(e_vec, seg_eff), seg_eff)
            s_fin = s_run + jnp.where(in_carry, s_c * fc, 0.0)

            acc = [a * fc for a in acc]
            for l in range(16):
                lane = jnp.full((16,), l, jnp.int32)
                e_l = _take(e_vec, lane)
                k_l = _take(keepf, lane)
                i = i0 + l
                xr = [xbufs[slot][i, pl.ds(16 * j, 16)] for j in range(8)]
                acc = [acc[j] * k_l + e_l * xr[j] for j in range(8)]
                local = jnp.clip(seg_eff[l] - SPT * wid, 0, SPT - 1)
                for j in range(8):
                    xsc[local, pl.ds(16 * j, 16)] = acc[j]
                sv2[local, :] = _take(s_fin, lane)

            return (seg_eff[15], jnp.full((16,), m_b[15], jnp.float32),
                    jnp.full((16,), s_fin[15], jnp.float32), tuple(acc))
        return body

    def process(k, slot, carry):
        cur_seg, m_c, s_c, acc, pe = carry
        start_k = chunk_start(k)
        lo = jnp.maximum(r0, pe)
        inner = lax.fori_loop(0, C // 16, group_body(start_k, lo, slot),
                              (cur_seg, m_c, s_c, acc))
        return inner + (start_k + C,)

    @pl.when(T > 0)
    def _prime():
        issue(0, 0)

    init = (jnp.int32(-1), neg_v, z16, tuple(z16 for _ in range(8)),
            jnp.int32(0))

    def pair_body(kk, carry):
        k0 = 2 * kk
        k1 = 2 * kk + 1

        @pl.when(k1 < T)
        def _():
            issue(k1, 1)
        wait(0)
        carry = process(k0, 0, carry)

        @pl.when(k1 + 1 < T)
        def _():
            issue(k1 + 1, 0)

        def do_k1(c):
            wait(1)
            return process(k1, 1, c)
        carry = lax.cond(k1 < T, do_k1, lambda c: c, carry)
        return carry

    lax.fori_loop(0, Thalf, pair_body, init)

    for sl in range(SPT):
        inv_v = 1.0 / (sv2[sl, pl.ds(0, 16)] + 1e-16)
        for j in range(8):
            xsc[sl, pl.ds(16 * j, 16)] = xsc[sl, pl.ds(16 * j, 16)] * inv_v
    pltpu.sync_copy(xsc, out_hbm.at[pl.ds(wid * SPT, SPT), :])


@functools.partial(
    pl.kernel,
    out_type=jax.ShapeDtypeStruct((G, D), jnp.float32),
    mesh=plsc.VectorSubcoreMesh(core_axis_name="c", subcore_axis_name="s"),
    scratch_types=[
        pltpu.VMEM((C, D), jnp.float32),
        pltpu.VMEM((C, D), jnp.float32),
        pltpu.VMEM((C,), jnp.int32),
        pltpu.VMEM((C,), jnp.int32),
        pltpu.VMEM((16,), jnp.int32),
        pltpu.VMEM((D,), jnp.float32),
        pltpu.VMEM((SPT, D), jnp.float32),
        pltpu.VMEM((SPT, 16), jnp.float32),
        pltpu.SemaphoreType.DMA((2, 2)),
    ],
    compiler_params=pltpu.CompilerParams(needs_layout_passes=False),
)
def _sc_pool_kernel(x_hbm, pb_hbm, bnd_hbm, gw_hbm, out_hbm,
                    xv0, xv1, pv0, pv1, bndv, gwv, xsc, sv2, sems):
    _sc_pool(x_hbm, pb_hbm, bnd_hbm, gw_hbm, out_hbm,
             xv0, xv1, pv0, pv1, bndv, gwv, xsc, sv2, sems)


def _heads_kernel(xs_ref, w1_ref, b1_ref, w2_ref, b2_ref,
                  w3_ref, b3_ref, w4_ref, b4_ref, de_ref, den_ref,
                  out1_ref, out2_ref):
    xs = xs_ref[...]
    h1 = jnp.maximum(jnp.dot(xs, w1_ref[...],
                             preferred_element_type=jnp.float32)
                     + b1_ref[0, :], 0.0)
    v_vec = jnp.dot(h1, w2_ref[...],
                    preferred_element_type=jnp.float32) + b2_ref[0, :]
    h2 = jnp.maximum(jnp.dot(xs, w3_ref[...],
                             preferred_element_type=jnp.float32)
                     + b3_ref[0, :], 0.0)
    v_norm = jnp.dot(h2, w4_ref[...],
                     preferred_element_type=jnp.float32) + b4_ref[0, :]
    de_mean = jnp.mean(de_ref[...], axis=0)
    den_mean = jnp.mean(den_ref[...], axis=0)
    out1_ref[...] = v_vec * de_mean[None, :]
    out2_ref[...] = v_norm * den_mean[None, :]


@jax.jit
def _run(x_clu, pb, bnds, gw,
         W1, b1, W2, b2, W3, b3, W4, b4, de, den):
    x_scene = _sc_pool_kernel(x_clu, pb, bnds, gw)
    return pl.pallas_call(
        _heads_kernel,
        out_shape=[jax.ShapeDtypeStruct((G, 6), jnp.float32),
                   jax.ShapeDtypeStruct((G, 1), jnp.float32)],
    )(x_scene, W1, b1.reshape(1, D), W2, b2.reshape(1, 6),
      W3, b3.reshape(1, D), W4, b4.reshape(1, 1), de, den)


def kernel(x_clu, mask_clu, batch_clu, dist_embedding, dist_embedding_norm,
           gate_W, gate_b, W1, b1, W2, b2, W3, b3, W4, b4):
    batchi = batch_clu.astype(jnp.int32)
    pb = batchi * 2 + mask_clu.astype(jnp.int32)
    qs = jnp.arange(NW + 1, dtype=jnp.int32) * SPT
    b33 = jnp.sum((batchi[:, None] < qs[None, :]).astype(jnp.int32),
                  axis=0).astype(jnp.int32)
    bnds = jnp.zeros((NW, 16), jnp.int32)
    bnds = bnds.at[:, 0].set(b33[:NW]).at[:, 1].set(b33[1:])
    out1, out2 = _run(x_clu, pb, bnds, gate_W[:, 0],
                      W1, b1, W2, b2, W3, b3, W4, b4,
                      dist_embedding, dist_embedding_norm)
    return out1, out2


# speedup vs baseline: 1.0864x; 1.0048x over previous
import functools

import jax
import jax.numpy as jnp
from jax import lax
from jax.experimental import pallas as pl
from jax.experimental.pallas import tpu as pltpu
from jax.experimental.pallas import tpu_sc as plsc

N = 100000
D = 128
G = 512
NC = 2
NS = 16
NW = NC * NS
SPT = G // NW
C = 112
NEG = -1e30


def _take(v, idx):
    return jnp.take_along_axis(v, idx, axis=0)


def _sc_pool(x_hbm, pb_hbm, bnd_hbm, gw_hbm, out_hbm,
             xv0, xv1, pv0, pv1, bndv, gwv, xsc, sv2, sems):
    wid = lax.axis_index("c") * NS + lax.axis_index("s")
    xbufs, pbufs = [xv0, xv1], [pv0, pv1]

    pltpu.sync_copy(bnd_hbm.at[wid], bndv)
    pltpu.sync_copy(gw_hbm, gwv)
    gws = [gwv[pl.ds(16 * j, 16)] for j in range(8)]

    z16 = jnp.zeros((16,), jnp.float32)
    for sl in range(SPT):
        for j in range(8):
            xsc[sl, pl.ds(16 * j, 16)] = z16
        sv2[sl, :] = z16

    bvec = bndv[pl.ds(0, 16)]
    r0 = bvec[0]
    r1 = bvec[1]
    a0 = (r0 // 8) * 8
    T = (r1 - a0 + C - 1) // C
    Thalf = (T + 1) // 2

    neg_v = jnp.full((16,), NEG, jnp.float32)
    iota = jnp.arange(16, dtype=jnp.int32)
    ones_v = jnp.full((16,), 1.0, jnp.float32)

    def chunk_start(k):
        return jnp.minimum(a0 + k * C, N - C)

    def issue(k, slot):
        s = chunk_start(k)
        pltpu.async_copy(x_hbm.at[pl.ds(s, C), :], xbufs[slot], sems.at[slot, 0])
        pltpu.async_copy(pb_hbm.at[pl.ds(s, C)], pbufs[slot], sems.at[slot, 1])

    def wait(slot):
        pltpu.make_async_copy(x_hbm.at[pl.ds(0, C), :], xbufs[slot],
                              sems.at[slot, 0]).wait()
        pltpu.make_async_copy(pb_hbm.at[pl.ds(0, C)], pbufs[slot],
                              sems.at[slot, 1]).wait()

    def seg_scan_max(x, seg):
        for k in (1, 2, 4, 8):
            src = jnp.maximum(iota - k, 0)
            ok = (iota >= k) & (_take(seg, src) == seg)
            x = jnp.where(ok, jnp.maximum(x, _take(x, src)), x)
        return x

    def seg_fill_back_max(x, seg):
        for k in (1, 2, 4, 8):
            src = jnp.minimum(iota + k, 15)
            ok = (iota + k <= 15) & (_take(seg, src) == seg)
            x = jnp.where(ok, jnp.maximum(x, _take(x, src)), x)
        return x

    def seg_scan_sum(x, seg):
        for k in (1, 2, 4, 8):
            src = jnp.maximum(iota - k, 0)
            ok = (iota >= k) & (_take(seg, src) == seg)
            x = jnp.where(ok, x + _take(x, src), x)
        return x

    def group_body(start_k, lo, slot):
        def body(gidx, carry):
            cur_seg, m_c, s_c, acc = carry
            i0 = gidx * 16
            rbase = start_k + i0

            pk = pbufs[slot][pl.ds(i0, 16)]
            seg_raw = pk >> 1
            mkf = (pk & 1).astype(jnp.float32)

            rvec = rbase + iota
            valid = (rvec >= lo) & (rvec < r1)

            fvi = jnp.min(jnp.where(valid, iota, 16))
            lvi = jnp.max(jnp.where(valid, iota, -1))
            slv_vec = _take(seg_raw, jnp.full((16,), jnp.maximum(lvi, 0),
                                              jnp.int32))
            cur_seg_v = jnp.full((16,), cur_seg, jnp.int32)
            seg_eff = jnp.where(valid, seg_raw,
                                jnp.where(iota < fvi, cur_seg_v, slv_vec))

            prev = jnp.where(iota == 0, cur_seg_v,
                             _take(seg_eff, jnp.maximum(iota - 1, 0)))
            run_start = seg_eff != prev
            keepf = jnp.where(run_start, 0.0, 1.0)

            g_vec = z16
            for l in range(16):
                i = i0 + l
                xr = [xbufs[slot][i, pl.ds(16 * j, 16)] for j in range(8)]
                p0 = ((xr[0] * gws[0] + xr[1] * gws[1])
                      + (xr[2] * gws[2] + xr[3] * gws[3]))
                p1 = ((xr[4] * gws[4] + xr[5] * gws[5])
                      + (xr[6] * gws[6] + xr[7] * gws[7]))
                g = jnp.sum(p0 + p1)
                g_vec = jnp.where(iota == l, g, g_vec)

            mk_on = (mkf > 0.0) & valid
            gm = jnp.where(mk_on, g_vec, neg_v)

            m_f = seg_scan_max(gm, seg_eff)
            in_carry = seg_eff == cur_seg_v
            m_f = jnp.where(in_carry, jnp.maximum(m_f, m_c), m_f)
            m_b = seg_fill_back_max(m_f, seg_eff)

            e_vec = jnp.exp(gm - m_b) * jnp.where(mk_on, ones_v, 0.0)

            mb0 = jnp.full((16,), m_b[0], jnp.float32)
            lane0_carry = seg_eff[0] == cur_seg
            fc = jnp.exp(jnp.where(lane0_carry, m_c - mb0, neg_v))

            s_run = seg_fill_back_max(seg_scan_sum(e_vec, seg_eff), seg_eff)
            s_fin = s_run + jnp.where(in_carry, s_c * fc, 0.0)

            acc = [a * fc for a in acc]
            for l in range(16):
                lane = jnp.full((16,), l, jnp.int32)
                e_l = _take(e_vec, lane)
                k_l = _take(keepf, lane)
                i = i0 + l
                xr = [xbufs[slot][i, pl.ds(16 * j, 16)] for j in range(8)]
                acc = [acc[j] * k_l + e_l * xr[j] for j in range(8)]
                local = jnp.clip(seg_eff[l] - SPT * wid, 0, SPT - 1)
                for j in range(8):
                    xsc[local, pl.ds(16 * j, 16)] = acc[j]
                sv2[local, :] = _take(s_fin, lane)

            return (seg_eff[15], jnp.full((16,), m_b[15], jnp.float32),
                    jnp.full((16,), s_fin[15], jnp.float32), tuple(acc))
        return body

    def process(k, slot, carry):
        cur_seg, m_c, s_c, acc, pe = carry
        start_k = chunk_start(k)
        lo = jnp.maximum(r0, pe)
        inner = lax.fori_loop(0, C // 16, group_body(start_k, lo, slot),
                              (cur_seg, m_c, s_c, acc))
        return inner + (start_k + C,)

    @pl.when(T > 0)
    def _prime():
        issue(0, 0)

    init = (jnp.int32(-1), neg_v, z16, tuple(z16 for _ in range(8)),
            jnp.int32(0))

    def pair_body(kk, carry):
        k0 = 2 * kk
        k1 = 2 * kk + 1

        @pl.when(k1 < T)
        def _():
            issue(k1, 1)
        wait(0)
        carry = process(k0, 0, carry)

        @pl.when(k1 + 1 < T)
        def _():
            issue(k1 + 1, 0)

        def do_k1(c):
            wait(1)
            return process(k1, 1, c)
        carry = lax.cond(k1 < T, do_k1, lambda c: c, carry)
        return carry

    lax.fori_loop(0, Thalf, pair_body, init)

    for sl in range(SPT):
        inv_v = 1.0 / (sv2[sl, pl.ds(0, 16)] + 1e-16)
        for j in range(8):
            xsc[sl, pl.ds(16 * j, 16)] = xsc[sl, pl.ds(16 * j, 16)] * inv_v
    pltpu.sync_copy(xsc, out_hbm.at[pl.ds(wid * SPT, SPT), :])


@functools.partial(
    pl.kernel,
    out_type=jax.ShapeDtypeStruct((G, D), jnp.float32),
    mesh=plsc.VectorSubcoreMesh(core_axis_name="c", subcore_axis_name="s"),
    scratch_types=[
        pltpu.VMEM((C, D), jnp.float32),
        pltpu.VMEM((C, D), jnp.float32),
        pltpu.VMEM((C,), jnp.int32),
        pltpu.VMEM((C,), jnp.int32),
        pltpu.VMEM((16,), jnp.int32),
        pltpu.VMEM((D,), jnp.float32),
        pltpu.VMEM((SPT, D), jnp.float32),
        pltpu.VMEM((SPT, 16), jnp.float32),
        pltpu.SemaphoreType.DMA((2, 2)),
    ],
    compiler_params=pltpu.CompilerParams(needs_layout_passes=False),
)
def _sc_pool_kernel(x_hbm, pb_hbm, bnd_hbm, gw_hbm, out_hbm,
                    xv0, xv1, pv0, pv1, bndv, gwv, xsc, sv2, sems):
    _sc_pool(x_hbm, pb_hbm, bnd_hbm, gw_hbm, out_hbm,
             xv0, xv1, pv0, pv1, bndv, gwv, xsc, sv2, sems)


def _heads_kernel(xs_ref, w1_ref, b1_ref, w2_ref, b2_ref,
                  w3_ref, b3_ref, w4_ref, b4_ref, de_ref, den_ref,
                  out1_ref, out2_ref):
    xs = xs_ref[...]
    h1 = jnp.maximum(jnp.dot(xs, w1_ref[...],
                             preferred_element_type=jnp.float32)
                     + b1_ref[0, :], 0.0)
    v_vec = jnp.dot(h1, w2_ref[...],
                    preferred_element_type=jnp.float32) + b2_ref[0, :]
    h2 = jnp.maximum(jnp.dot(xs, w3_ref[...],
                             preferred_element_type=jnp.float32)
                     + b3_ref[0, :], 0.0)
    v_norm = jnp.dot(h2, w4_ref[...],
                     preferred_element_type=jnp.float32) + b4_ref[0, :]
    de_mean = jnp.mean(de_ref[...], axis=0)
    den_mean = jnp.mean(den_ref[...], axis=0)
    out1_ref[...] = v_vec * de_mean[None, :]
    out2_ref[...] = v_norm * den_mean[None, :]


@jax.jit
def _run(x_clu, pb, bnds, gw,
         W1, b1, W2, b2, W3, b3, W4, b4, de, den):
    x_scene = _sc_pool_kernel(x_clu, pb, bnds, gw)
    return pl.pallas_call(
        _heads_kernel,
        out_shape=[jax.ShapeDtypeStruct((G, 6), jnp.float32),
                   jax.ShapeDtypeStruct((G, 1), jnp.float32)],
    )(x_scene, W1, b1.reshape(1, D), W2, b2.reshape(1, 6),
      W3, b3.reshape(1, D), W4, b4.reshape(1, 1), de, den)


def kernel(x_clu, mask_clu, batch_clu, dist_embedding, dist_embedding_norm,
           gate_W, gate_b, W1, b1, W2, b2, W3, b3, W4, b4):
    batchi = batch_clu.astype(jnp.int32)
    pb = batchi * 2 + mask_clu.astype(jnp.int32)
    qs = jnp.arange(NW + 1, dtype=jnp.int32) * SPT
    b33 = jnp.sum((batchi[:, None] < qs[None, :]).astype(jnp.int32),
                  axis=0).astype(jnp.int32)
    bnds = jnp.zeros((NW, 16), jnp.int32)
    bnds = bnds.at[:, 0].set(b33[:NW]).at[:, 1].set(b33[1:])
    out1, out2 = _run(x_clu, pb, bnds, gate_W[:, 0],
                      W1, b1, W2, b2, W3, b3, W4, b4,
                      dist_embedding, dist_embedding_norm)
    return out1, out2


# speedup vs baseline: 1.1033x; 1.0156x over previous
import functools

import jax
import jax.numpy as jnp
from jax import lax
from jax.experimental import pallas as pl
from jax.experimental.pallas import tpu as pltpu
from jax.experimental.pallas import tpu_sc as plsc

N = 100000
D = 128
G = 512
NC = 2
NS = 16
NW = NC * NS
SPT = G // NW
C = 80
NEG = -1e30


def _take(v, idx):
    return jnp.take_along_axis(v, idx, axis=0)


def _sc_pool(x_hbm, pb_hbm, bnd_hbm, gw_hbm, out_hbm,
             xv0, xv1, pv0, pv1, bndv, gwv, xsc, sv2, sems):
    wid = lax.axis_index("c") * NS + lax.axis_index("s")
    xbufs, pbufs = [xv0, xv1], [pv0, pv1]

    pltpu.sync_copy(bnd_hbm.at[wid], bndv)
    pltpu.sync_copy(gw_hbm, gwv)
    gws = [gwv[pl.ds(16 * j, 16)] for j in range(8)]

    z16 = jnp.zeros((16,), jnp.float32)
    for sl in range(SPT):
        for j in range(8):
            xsc[sl, pl.ds(16 * j, 16)] = z16
        sv2[sl, :] = z16

    bvec = bndv[pl.ds(0, 16)]
    r0 = bvec[0]
    r1 = bvec[1]
    a0 = (r0 // 8) * 8
    T = (r1 - a0 + C - 1) // C
    Thalf = (T + 1) // 2

    neg_v = jnp.full((16,), NEG, jnp.float32)
    iota = jnp.arange(16, dtype=jnp.int32)
    ones_v = jnp.full((16,), 1.0, jnp.float32)

    def chunk_start(k):
        return jnp.minimum(a0 + k * C, N - C)

    def issue(k, slot):
        s = chunk_start(k)
        pltpu.async_copy(x_hbm.at[pl.ds(s, C), :], xbufs[slot], sems.at[slot, 0])
        pltpu.async_copy(pb_hbm.at[pl.ds(s, C)], pbufs[slot], sems.at[slot, 1])

    def wait(slot):
        pltpu.make_async_copy(x_hbm.at[pl.ds(0, C), :], xbufs[slot],
                              sems.at[slot, 0]).wait()
        pltpu.make_async_copy(pb_hbm.at[pl.ds(0, C)], pbufs[slot],
                              sems.at[slot, 1]).wait()

    def seg_scan_max(x, seg):
        for k in (1, 2, 4, 8):
            src = jnp.maximum(iota - k, 0)
            ok = (iota >= k) & (_take(seg, src) == seg)
            x = jnp.where(ok, jnp.maximum(x, _take(x, src)), x)
        return x

    def seg_fill_back_max(x, seg):
        for k in (1, 2, 4, 8):
            src = jnp.minimum(iota + k, 15)
            ok = (iota + k <= 15) & (_take(seg, src) == seg)
            x = jnp.where(ok, jnp.maximum(x, _take(x, src)), x)
        return x

    def seg_scan_sum(x, seg):
        for k in (1, 2, 4, 8):
            src = jnp.maximum(iota - k, 0)
            ok = (iota >= k) & (_take(seg, src) == seg)
            x = jnp.where(ok, x + _take(x, src), x)
        return x

    def group_body(start_k, lo, slot):
        def body(gidx, carry):
            cur_seg, m_c, s_c, acc = carry
            i0 = gidx * 16
            rbase = start_k + i0

            pk = pbufs[slot][pl.ds(i0, 16)]
            seg_raw = pk >> 1
            mkf = (pk & 1).astype(jnp.float32)

            rvec = rbase + iota
            valid = (rvec >= lo) & (rvec < r1)

            fvi = jnp.min(jnp.where(valid, iota, 16))
            lvi = jnp.max(jnp.where(valid, iota, -1))
            slv_vec = _take(seg_raw, jnp.full((16,), jnp.maximum(lvi, 0),
                                              jnp.int32))
            cur_seg_v = jnp.full((16,), cur_seg, jnp.int32)
            seg_eff = jnp.where(valid, seg_raw,
                                jnp.where(iota < fvi, cur_seg_v, slv_vec))

            prev = jnp.where(iota == 0, cur_seg_v,
                             _take(seg_eff, jnp.maximum(iota - 1, 0)))
            run_start = seg_eff != prev
            keepf = jnp.where(run_start, 0.0, 1.0)

            g_vec = z16
            for l in range(16):
                i = i0 + l
                xr = [xbufs[slot][i, pl.ds(16 * j, 16)] for j in range(8)]
                p0 = ((xr[0] * gws[0] + xr[1] * gws[1])
                      + (xr[2] * gws[2] + xr[3] * gws[3]))
                p1 = ((xr[4] * gws[4] + xr[5] * gws[5])
                      + (xr[6] * gws[6] + xr[7] * gws[7]))
                g = jnp.sum(p0 + p1)
                g_vec = jnp.where(iota == l, g, g_vec)

            mk_on = (mkf > 0.0) & valid
            gm = jnp.where(mk_on, g_vec, neg_v)

            m_f = seg_scan_max(gm, seg_eff)
            in_carry = seg_eff == cur_seg_v
            m_f = jnp.where(in_carry, jnp.maximum(m_f, m_c), m_f)
            m_b = seg_fill_back_max(m_f, seg_eff)

            e_vec = jnp.exp(gm - m_b) * jnp.where(mk_on, ones_v, 0.0)

            mb0 = jnp.full((16,), m_b[0], jnp.float32)
            lane0_carry = seg_eff[0] == cur_seg
            fc = jnp.exp(jnp.where(lane0_carry, m_c - mb0, neg_v))

            s_run = seg_fill_back_max(seg_scan_sum(e_vec, seg_eff), seg_eff)
            s_fin = s_run + jnp.where(in_carry, s_c * fc, 0.0)

            acc = [a * fc for a in acc]
            for l in range(16):
                lane = jnp.full((16,), l, jnp.int32)
                e_l = _take(e_vec, lane)
                k_l = _take(keepf, lane)
                i = i0 + l
                xr = [xbufs[slot][i, pl.ds(16 * j, 16)] for j in range(8)]
                acc = [acc[j] * k_l + e_l * xr[j] for j in range(8)]
                local = jnp.clip(seg_eff[l] - SPT * wid, 0, SPT - 1)
                for j in range(8):
                    xsc[local, pl.ds(16 * j, 16)] = acc[j]
                sv2[local, :] = _take(s_fin, lane)

            return (seg_eff[15], jnp.full((16,), m_b[15], jnp.float32),
                    jnp.full((16,), s_fin[15], jnp.float32), tuple(acc))
        return body

    def process(k, slot, carry):
        cur_seg, m_c, s_c, acc, pe = carry
        start_k = chunk_start(k)
        lo = jnp.maximum(r0, pe)
        inner = lax.fori_loop(0, C // 16, group_body(start_k, lo, slot),
                              (cur_seg, m_c, s_c, acc))
        return inner + (start_k + C,)

    @pl.when(T > 0)
    def _prime():
        issue(0, 0)

    init = (jnp.int32(-1), neg_v, z16, tuple(z16 for _ in range(8)),
            jnp.int32(0))

    def pair_body(kk, carry):
        k0 = 2 * kk
        k1 = 2 * kk + 1

        @pl.when(k1 < T)
        def _():
            issue(k1, 1)
        wait(0)
        carry = process(k0, 0, carry)

        @pl.when(k1 + 1 < T)
        def _():
            issue(k1 + 1, 0)

        def do_k1(c):
            wait(1)
            return process(k1, 1, c)
        carry = lax.cond(k1 < T, do_k1, lambda c: c, carry)
        return carry

    lax.fori_loop(0, Thalf, pair_body, init)

    for sl in range(SPT):
        inv_v = 1.0 / (sv2[sl, pl.ds(0, 16)] + 1e-16)
        for j in range(8):
            xsc[sl, pl.ds(16 * j, 16)] = xsc[sl, pl.ds(16 * j, 16)] * inv_v
    pltpu.sync_copy(xsc, out_hbm.at[pl.ds(wid * SPT, SPT), :])


@functools.partial(
    pl.kernel,
    out_type=jax.ShapeDtypeStruct((G, D), jnp.float32),
    mesh=plsc.VectorSubcoreMesh(core_axis_name="c", subcore_axis_name="s"),
    scratch_types=[
        pltpu.VMEM((C, D), jnp.float32),
        pltpu.VMEM((C, D), jnp.float32),
        pltpu.VMEM((C,), jnp.int32),
        pltpu.VMEM((C,), jnp.int32),
        pltpu.VMEM((16,), jnp.int32),
        pltpu.VMEM((D,), jnp.float32),
        pltpu.VMEM((SPT, D), jnp.float32),
        pltpu.VMEM((SPT, 16), jnp.float32),
        pltpu.SemaphoreType.DMA((2, 2)),
    ],
    compiler_params=pltpu.CompilerParams(needs_layout_passes=False),
)
def _sc_pool_kernel(x_hbm, pb_hbm, bnd_hbm, gw_hbm, out_hbm,
                    xv0, xv1, pv0, pv1, bndv, gwv, xsc, sv2, sems):
    _sc_pool(x_hbm, pb_hbm, bnd_hbm, gw_hbm, out_hbm,
             xv0, xv1, pv0, pv1, bndv, gwv, xsc, sv2, sems)


def _heads_kernel(xs_ref, w1_ref, b1_ref, w2_ref, b2_ref,
                  w3_ref, b3_ref, w4_ref, b4_ref, de_ref, den_ref,
                  out1_ref, out2_ref):
    xs = xs_ref[...]
    h1 = jnp.maximum(jnp.dot(xs, w1_ref[...],
                             preferred_element_type=jnp.float32)
                     + b1_ref[0, :], 0.0)
    v_vec = jnp.dot(h1, w2_ref[...],
                    preferred_element_type=jnp.float32) + b2_ref[0, :]
    h2 = jnp.maximum(jnp.dot(xs, w3_ref[...],
                             preferred_element_type=jnp.float32)
                     + b3_ref[0, :], 0.0)
    v_norm = jnp.dot(h2, w4_ref[...],
                     preferred_element_type=jnp.float32) + b4_ref[0, :]
    de_mean = jnp.mean(de_ref[...], axis=0)
    den_mean = jnp.mean(den_ref[...], axis=0)
    out1_ref[...] = v_vec * de_mean[None, :]
    out2_ref[...] = v_norm * den_mean[None, :]


@jax.jit
def _run(x_clu, pb, bnds, gw,
         W1, b1, W2, b2, W3, b3, W4, b4, de, den):
    x_scene = _sc_pool_kernel(x_clu, pb, bnds, gw)
    return pl.pallas_call(
        _heads_kernel,
        out_shape=[jax.ShapeDtypeStruct((G, 6), jnp.float32),
                   jax.ShapeDtypeStruct((G, 1), jnp.float32)],
    )(x_scene, W1, b1.reshape(1, D), W2, b2.reshape(1, 6),
      W3, b3.reshape(1, D), W4, b4.reshape(1, 1), de, den)


def kernel(x_clu, mask_clu, batch_clu, dist_embedding, dist_embedding_norm,
           gate_W, gate_b, W1, b1, W2, b2, W3, b3, W4, b4):
    batchi = batch_clu.astype(jnp.int32)
    pb = batchi * 2 + mask_clu.astype(jnp.int32)
    qs = jnp.arange(NW + 1, dtype=jnp.int32) * SPT
    b33 = jnp.sum((batchi[:, None] < qs[None, :]).astype(jnp.int32),
                  axis=0).astype(jnp.int32)
    bnds = jnp.zeros((NW, 16), jnp.int32)
    bnds = bnds.at[:, 0].set(b33[:NW]).at[:, 1].set(b33[1:])
    out1, out2 = _run(x_clu, pb, bnds, gate_W[:, 0],
                      W1, b1, W2, b2, W3, b3, W4, b4,
                      dist_embedding, dist_embedding_norm)
    return out1, out2


# speedup vs baseline: 1.1067x; 1.0031x over previous
import functools

import jax
import jax.numpy as jnp
from jax import lax
from jax.experimental import pallas as pl
from jax.experimental.pallas import tpu as pltpu
from jax.experimental.pallas import tpu_sc as plsc

N = 100000
D = 128
G = 512
NC = 2
NS = 16
NW = NC * NS
SPT = G // NW
C = 48
NEG = -1e30


def _take(v, idx):
    return jnp.take_along_axis(v, idx, axis=0)


def _sc_pool(x_hbm, pb_hbm, bnd_hbm, gw_hbm, out_hbm,
             xv0, xv1, pv0, pv1, bndv, gwv, xsc, sv2, sems):
    wid = lax.axis_index("c") * NS + lax.axis_index("s")
    xbufs, pbufs = [xv0, xv1], [pv0, pv1]

    pltpu.sync_copy(bnd_hbm.at[wid], bndv)
    pltpu.sync_copy(gw_hbm, gwv)
    gws = [gwv[pl.ds(16 * j, 16)] for j in range(8)]

    z16 = jnp.zeros((16,), jnp.float32)
    for sl in range(SPT):
        for j in range(8):
            xsc[sl, pl.ds(16 * j, 16)] = z16
        sv2[sl, :] = z16

    bvec = bndv[pl.ds(0, 16)]
    r0 = bvec[0]
    r1 = bvec[1]
    a0 = (r0 // 8) * 8
    T = (r1 - a0 + C - 1) // C
    Thalf = (T + 1) // 2

    neg_v = jnp.full((16,), NEG, jnp.float32)
    iota = jnp.arange(16, dtype=jnp.int32)
    ones_v = jnp.full((16,), 1.0, jnp.float32)

    def chunk_start(k):
        return jnp.minimum(a0 + k * C, N - C)

    def issue(k, slot):
        s = chunk_start(k)
        pltpu.async_copy(x_hbm.at[pl.ds(s, C), :], xbufs[slot], sems.at[slot, 0])
        pltpu.async_copy(pb_hbm.at[pl.ds(s, C)], pbufs[slot], sems.at[slot, 1])

    def wait(slot):
        pltpu.make_async_copy(x_hbm.at[pl.ds(0, C), :], xbufs[slot],
                              sems.at[slot, 0]).wait()
        pltpu.make_async_copy(pb_hbm.at[pl.ds(0, C)], pbufs[slot],
                              sems.at[slot, 1]).wait()

    def seg_scan_max(x, seg):
        for k in (1, 2, 4, 8):
            src = jnp.maximum(iota - k, 0)
            ok = (iota >= k) & (_take(seg, src) == seg)
            x = jnp.where(ok, jnp.maximum(x, _take(x, src)), x)
        return x

    def seg_fill_back_max(x, seg):
        for k in (1, 2, 4, 8):
            src = jnp.minimum(iota + k, 15)
            ok = (iota + k <= 15) & (_take(seg, src) == seg)
            x = jnp.where(ok, jnp.maximum(x, _take(x, src)), x)
        return x

    def seg_scan_sum(x, seg):
        for k in (1, 2, 4, 8):
            src = jnp.maximum(iota - k, 0)
            ok = (iota >= k) & (_take(seg, src) == seg)
            x = jnp.where(ok, x + _take(x, src), x)
        return x

    def group_body(start_k, lo, slot):
        def body(gidx, carry):
            cur_seg, m_c, s_c, acc = carry
            i0 = gidx * 16
            rbase = start_k + i0

            pk = pbufs[slot][pl.ds(i0, 16)]
            seg_raw = pk >> 1
            mkf = (pk & 1).astype(jnp.float32)

            rvec = rbase + iota
            valid = (rvec >= lo) & (rvec < r1)

            fvi = jnp.min(jnp.where(valid, iota, 16))
            lvi = jnp.max(jnp.where(valid, iota, -1))
            slv_vec = _take(seg_raw, jnp.full((16,), jnp.maximum(lvi, 0),
                                              jnp.int32))
            cur_seg_v = jnp.full((16,), cur_seg, jnp.int32)
            seg_eff = jnp.where(valid, seg_raw,
                                jnp.where(iota < fvi, cur_seg_v, slv_vec))

            prev = jnp.where(iota == 0, cur_seg_v,
                             _take(seg_eff, jnp.maximum(iota - 1, 0)))
            run_start = seg_eff != prev
            keepf = jnp.where(run_start, 0.0, 1.0)

            g_vec = z16
            for l in range(16):
                i = i0 + l
                xr = [xbufs[slot][i, pl.ds(16 * j, 16)] for j in range(8)]
                p0 = ((xr[0] * gws[0] + xr[1] * gws[1])
                      + (xr[2] * gws[2] + xr[3] * gws[3]))
                p1 = ((xr[4] * gws[4] + xr[5] * gws[5])
                      + (xr[6] * gws[6] + xr[7] * gws[7]))
                g = jnp.sum(p0 + p1)
                g_vec = jnp.where(iota == l, g, g_vec)

            mk_on = (mkf > 0.0) & valid
            gm = jnp.where(mk_on, g_vec, neg_v)

            m_f = seg_scan_max(gm, seg_eff)
            in_carry = seg_eff == cur_seg_v
            m_f = jnp.where(in_carry, jnp.maximum(m_f, m_c), m_f)
            m_b = seg_fill_back_max(m_f, seg_eff)

            e_vec = jnp.exp(gm - m_b) * jnp.where(mk_on, ones_v, 0.0)

            mb0 = jnp.full((16,), m_b[0], jnp.float32)
            lane0_carry = seg_eff[0] == cur_seg
            fc = jnp.exp(jnp.where(lane0_carry, m_c - mb0, neg_v))

            s_run = seg_fill_back_max(seg_scan_sum(e_vec, seg_eff), seg_eff)
            s_fin = s_run + jnp.where(in_carry, s_c * fc, 0.0)

            acc = [a * fc for a in acc]
            for l in range(16):
                lane = jnp.full((16,), l, jnp.int32)
                e_l = _take(e_vec, lane)
                k_l = _take(keepf, lane)
                i = i0 + l
                xr = [xbufs[slot][i, pl.ds(16 * j, 16)] for j in range(8)]
                acc = [acc[j] * k_l + e_l * xr[j] for j in range(8)]
                local = jnp.clip(seg_eff[l] - SPT * wid, 0, SPT - 1)
                for j in range(8):
                    xsc[local, pl.ds(16 * j, 16)] = acc[j]
                sv2[local, :] = _take(s_fin, lane)

            return (seg_eff[15], jnp.full((16,), m_b[15], jnp.float32),
                    jnp.full((16,), s_fin[15], jnp.float32), tuple(acc))
        return body

    def process(k, slot, carry):
        cur_seg, m_c, s_c, acc, pe = carry
        start_k = chunk_start(k)
        lo = jnp.maximum(r0, pe)
        inner = lax.fori_loop(0, C // 16, group_body(start_k, lo, slot),
                              (cur_seg, m_c, s_c, acc))
        return inner + (start_k + C,)

    @pl.when(T > 0)
    def _prime():
        issue(0, 0)

    init = (jnp.int32(-1), neg_v, z16, tuple(z16 for _ in range(8)),
            jnp.int32(0))

    def pair_body(kk, carry):
        k0 = 2 * kk
        k1 = 2 * kk + 1

        @pl.when(k1 < T)
        def _():
            issue(k1, 1)
        wait(0)
        carry = process(k0, 0, carry)

        @pl.when(k1 + 1 < T)
        def _():
            issue(k1 + 1, 0)

        def do_k1(c):
            wait(1)
            return process(k1, 1, c)
        carry = lax.cond(k1 < T, do_k1, lambda c: c, carry)
        return carry

    lax.fori_loop(0, Thalf, pair_body, init)

    for sl in range(SPT):
        inv_v = 1.0 / (sv2[sl, pl.ds(0, 16)] + 1e-16)
        for j in range(8):
            xsc[sl, pl.ds(16 * j, 16)] = xsc[sl, pl.ds(16 * j, 16)] * inv_v
    pltpu.sync_copy(xsc, out_hbm.at[pl.ds(wid * SPT, SPT), :])


@functools.partial(
    pl.kernel,
    out_type=jax.ShapeDtypeStruct((G, D), jnp.float32),
    mesh=plsc.VectorSubcoreMesh(core_axis_name="c", subcore_axis_name="s"),
    scratch_types=[
        pltpu.VMEM((C, D), jnp.float32),
        pltpu.VMEM((C, D), jnp.float32),
        pltpu.VMEM((C,), jnp.int32),
        pltpu.VMEM((C,), jnp.int32),
        pltpu.VMEM((16,), jnp.int32),
        pltpu.VMEM((D,), jnp.float32),
        pltpu.VMEM((SPT, D), jnp.float32),
        pltpu.VMEM((SPT, 16), jnp.float32),
        pltpu.SemaphoreType.DMA((2, 2)),
    ],
    compiler_params=pltpu.CompilerParams(needs_layout_passes=False),
)
def _sc_pool_kernel(x_hbm, pb_hbm, bnd_hbm, gw_hbm, out_hbm,
                    xv0, xv1, pv0, pv1, bndv, gwv, xsc, sv2, sems):
    _sc_pool(x_hbm, pb_hbm, bnd_hbm, gw_hbm, out_hbm,
             xv0, xv1, pv0, pv1, bndv, gwv, xsc, sv2, sems)


def _heads_kernel(xs_ref, w1_ref, b1_ref, w2_ref, b2_ref,
                  w3_ref, b3_ref, w4_ref, b4_ref, de_ref, den_ref,
                  out1_ref, out2_ref):
    xs = xs_ref[...]
    h1 = jnp.maximum(jnp.dot(xs, w1_ref[...],
                             preferred_element_type=jnp.float32)
                     + b1_ref[0, :], 0.0)
    v_vec = jnp.dot(h1, w2_ref[...],
                    preferred_element_type=jnp.float32) + b2_ref[0, :]
    h2 = jnp.maximum(jnp.dot(xs, w3_ref[...],
                             preferred_element_type=jnp.float32)
                     + b3_ref[0, :], 0.0)
    v_norm = jnp.dot(h2, w4_ref[...],
                     preferred_element_type=jnp.float32) + b4_ref[0, :]
    de_mean = jnp.mean(de_ref[...], axis=0)
    den_mean = jnp.mean(den_ref[...], axis=0)
    out1_ref[...] = v_vec * de_mean[None, :]
    out2_ref[...] = v_norm * den_mean[None, :]


@jax.jit
def _run(x_clu, pb, bnds, gw,
         W1, b1, W2, b2, W3, b3, W4, b4, de, den):
    x_scene = _sc_pool_kernel(x_clu, pb, bnds, gw)
    return pl.pallas_call(
        _heads_kernel,
        out_shape=[jax.ShapeDtypeStruct((G, 6), jnp.float32),
                   jax.ShapeDtypeStruct((G, 1), jnp.float32)],
    )(x_scene, W1, b1.reshape(1, D), W2, b2.reshape(1, 6),
      W3, b3.reshape(1, D), W4, b4.reshape(1, 1), de, den)


def kernel(x_clu, mask_clu, batch_clu, dist_embedding, dist_embedding_norm,
           gate_W, gate_b, W1, b1, W2, b2, W3, b3, W4, b4):
    batchi = batch_clu.astype(jnp.int32)
    pb = batchi * 2 + mask_clu.astype(jnp.int32)
    qs = jnp.arange(NW + 1, dtype=jnp.int32) * SPT
    b33 = jnp.sum((batchi[:, None] < qs[None, :]).astype(jnp.int32),
                  axis=0).astype(jnp.int32)
    bnds = jnp.zeros((NW, 16), jnp.int32)
    bnds = bnds.at[:, 0].set(b33[:NW]).at[:, 1].set(b33[1:])
    out1, out2 = _run(x_clu, pb, bnds, gate_W[:, 0],
                      W1, b1, W2, b2, W3, b3, W4, b4,
                      dist_embedding, dist_embedding_norm)
    return out1, out2
